# baseline (speedup 1.0000x reference)
import numpy as np
import jax
import jax.numpy as jnp
from jax.experimental import pallas as pl
from jax.experimental.pallas import tpu as pltpu

_F32 = jnp.float32
_BF = jnp.bfloat16
_LEV = [(16, 128, 128), (8, 64, 64), (4, 32, 32), (2, 16, 16)]
_SD = [d + 2 for (d, h, w) in _LEV]
_SH = [h + 2 for (d, h, w) in _LEV]
_Ps = [sd * sh * 128 for sd, sh in zip(_SD, _SH)]
_Gs = [sh * 128 + 128 for sh in _SH]
_NT = [p + 2 * g for p, g in zip(_Ps, _Gs)]
_NTS = [p + g for p, g in zip(_Ps, _Gs)]


def _taps9(lvl):
    sh = _SH[lvl]
    return [(kz - 1) * sh * 128 + (ky - 1) * 128
            for kz in range(3) for ky in range(3)]


def _down_taps(lvl):
    sh = _SH[lvl]
    out = []
    for kz in range(3):
        for ky in range(3):
            for kx in range(3):
                rz, qz = kz & 1, kz >> 1
                ry, qy = ky & 1, ky >> 1
                if kx == 1:
                    slot = rz * 4 + ry * 2
                elif kx == 2:
                    slot = rz * 4 + ry * 2 + 1
                else:
                    slot = 8 + rz * 2 + ry
                out.append((slot, qz * sh * 128 + qy * 128))
    return out


def _aux_np(lvl):
    nt = _NT[lvl]
    d, h, w = _LEV[lvl]
    lanes = np.arange(nt) % 128
    m = np.zeros((8, nt), np.float32)
    m[0] = (lanes != 0).astype(np.float32)
    m[1] = (lanes != 127).astype(np.float32)
    s = np.zeros((_SD[lvl], _SH[lvl], 128), np.float32)
    s[1:-1, 1:-1, :w] = 1.0
    m[2, _Gs[lvl]:_Gs[lvl] + _Ps[lvl]] = s.reshape(-1)
    return m


_AUX = [_aux_np(l) for l in range(4)]


def _mm(w, x):
    return jax.lax.dot_general(w, x, (((1,), (0,)), ((), ())),
                               preferred_element_type=_F32)


def _build_shifts(s3_ref, aux_ref, c, nt):
    s3_ref[0:c, 0:1] = jnp.zeros((c, 1), _BF)
    s3_ref[0:c, 1:nt] = s3_ref[c:2 * c, 0:nt - 1] * aux_ref[0:1, 1:nt]
    s3_ref[2 * c:3 * c, nt - 1:nt] = jnp.zeros((c, 1), _BF)
    s3_ref[2 * c:3 * c, 0:nt - 1] = (s3_ref[c:2 * c, 1:nt]
                                     * aux_ref[1:2, 0:nt - 1])


def _zero_guards(s3_ref, c, lvl):
    g, p = _Gs[lvl], _Ps[lvl]
    s3_ref[c:2 * c, 0:g] = jnp.zeros((c, g), _BF)
    s3_ref[c:2 * c, g + p:] = jnp.zeros((c, g), _BF)


def _lvl0_body(x_ref, aux_ref, wa_ref, ba_ref, wb_ref, bb_ref, o_ref, s3_ref):
    g, p, nt = _Gs[0], _Ps[0], _NT[0]
    nc = 5

    def mask_fn(c0, cl):
        return aux_ref[2:3, g + c0:g + c0 + cl].astype(_F32)

    s3_ref[4:8, :] = x_ref[...]
    s3_ref[0:4, 0:1] = jnp.zeros((4, 1), _BF)
    s3_ref[0:4, 1:nt] = s3_ref[4:8, 0:nt - 1] * aux_ref[0:1, 1:nt]
    s3_ref[8:12, nt - 1:nt] = jnp.zeros((4, 1), _BF)
    s3_ref[8:12, 0:nt - 1] = s3_ref[4:8, 1:nt] * aux_ref[1:2, 0:nt - 1]
    s3_ref[16:32, 0:g] = jnp.zeros((16, g), _BF)
    s3_ref[16:32, g + p:] = jnp.zeros((16, g), _BF)

    cl = p // nc
    taps = _taps9(0)
    for ci in range(nc):
        c0 = ci * cl
        acc = jnp.zeros((16, cl), _F32)
        for j, off in enumerate(taps):
            acc = acc + _mm(wa_ref[j],
                            s3_ref[0:12, g + off + c0:g + off + c0 + cl])
        t = jnp.maximum(acc + ba_ref[...], 0.0) * mask_fn(c0, cl)
        s3_ref[16:32, g + c0:g + c0 + cl] = t.astype(_BF)

    _build_shifts(s3_ref, aux_ref, 16, nt)
    for ci in range(nc):
        c0 = ci * cl
        acc = jnp.zeros((16, cl), _F32)
        for j, off in enumerate(taps):
            acc = acc + _mm(wb_ref[j],
                            s3_ref[0:48, g + off + c0:g + off + c0 + cl])
        t = jnp.maximum(acc + bb_ref[...], 0.0) * mask_fn(c0, cl)
        o_ref[:, c0:c0 + cl] = t.astype(_BF)


def _make_down_body(lvl, c_in, c_out, nc):
    g, p = _Gs[lvl], _Ps[lvl]
    dtaps = _down_taps(lvl)

    def body(yp_ref, mp_ref, aux_ref, wd_ref, bd_ref, oy_ref, om_ref):
        cl = p // nc
        for ci in range(nc):
            c0 = ci * cl
            msum = jnp.zeros((1, cl), _F32)
            for (slot, off) in dtaps:
                msum = msum + mp_ref[slot:slot + 1,
                                     off + c0:off + c0 + cl].astype(_F32)
            nm = (jnp.where(msum > 0.0, 1.0, 0.0)
                  * aux_ref[2:3, g + c0:g + c0 + cl].astype(_F32))
            om_ref[:, c0:c0 + cl] = jnp.broadcast_to(
                nm.astype(_BF), (8, cl))
            acc = jnp.zeros((c_out, cl), _F32)
            for k, (slot, off) in enumerate(dtaps):
                acc = acc + _mm(wd_ref[k],
                                yp_ref[slot][:, off + c0:off + c0 + cl])
            val = jnp.maximum(acc + bd_ref[...], 0.0) * nm
            oy_ref[:, c0:c0 + cl] = val.astype(_BF)

    return body


def _make_subm_body(lvl, c, n_subm, nc, has_ex):
    g, p, nt = _Gs[lvl], _Ps[lvl], _NT[lvl]
    taps = _taps9(lvl)

    def body(*refs):
        x_ref, nm_ref, aux_ref = refs[:3]
        wbs = [(refs[3 + 2 * i], refs[4 + 2 * i]) for i in range(n_subm)]
        pos = 3 + 2 * n_subm
        if has_ex:
            wex_ref, bex_ref = refs[pos], refs[pos + 1]
            pos += 2
        oy_ref = refs[pos]
        s3_ref = refs[pos + 1]

        def nm_fn(c0, cl):
            return nm_ref[0:1, c0:c0 + cl].astype(_F32)

        _zero_guards(s3_ref, c, lvl)
        s3_ref[c:2 * c, g:g + p] = x_ref[...]
        cl = p // nc
        for li, (w9_ref, b_ref) in enumerate(wbs):
            _build_shifts(s3_ref, aux_ref, c, nt)
            last = li == n_subm - 1
            for ci in range(nc):
                c0 = ci * cl
                acc = jnp.zeros((c, cl), _F32)
                for j, off in enumerate(taps):
                    acc = acc + _mm(
                        w9_ref[j],
                        s3_ref[0:3 * c, g + off + c0:g + off + c0 + cl])
                val = jnp.maximum(acc + b_ref[...], 0.0) * nm_fn(c0, cl)
                if last and not has_ex:
                    oy_ref[:, c0:c0 + cl] = val.astype(_BF)
                else:
                    s3_ref[3 * c:4 * c, g + c0:g + c0 + cl] = val.astype(_BF)
            if not (last and not has_ex):
                s3_ref[c:2 * c, g:g + p] = s3_ref[3 * c:4 * c, g:g + p]
        if has_ex:
            for ci in range(2):
                c0 = ci * (p // 2)
                cle = p // 2
                acc = _mm(wex_ref[...], s3_ref[c:2 * c, g + c0:g + c0 + cle])
                oy_ref[:, c0:c0 + cle] = (
                    jnp.maximum(acc + bex_ref[...], 0.0) * nm_fn(c0, cle))

    return body


def _prep_w27(w, gg, eps=1e-3):
    s = gg / jnp.sqrt(1.0 + eps)
    w = w * s[:, None, None, None, None]
    co, ci = w.shape[0], w.shape[1]
    return jnp.transpose(w, (2, 3, 4, 0, 1)).reshape(27, co, ci).astype(_BF)


def _prep_w9(w, gg, eps=1e-3):
    w27 = _prep_w27(w, gg, eps)
    co, ci = w27.shape[1], w27.shape[2]
    return w27.reshape(9, 3, co, ci).transpose(0, 2, 1, 3).reshape(
        9, co, 3 * ci)


def _parity_pack(y_flat, m_flat, lvl):
    C = y_flat.shape[0]
    nl = lvl + 1
    w_cur = _LEV[lvl][2]
    y4 = y_flat.reshape(C, _SD[lvl], _SH[lvl], 128)[:, :, :, :w_cur]
    m4 = m_flat.reshape(1, _SD[lvl], _SH[lvl], 128)[:, :, :, :w_cur]
    gn = _Gs[nl]

    def emb(a4):
        cc = a4.shape[0]
        a4 = jnp.pad(a4, ((0, 0), (1, 0), (1, 0), (0, 128 - a4.shape[3])))
        return jnp.pad(a4.reshape(cc, -1), ((0, 0), (0, gn)))

    ys, ms = [], []
    for rz in (0, 1):
        for ry in (0, 1):
            for rx in (0, 1):
                ys.append(emb(y4[:, rz::2, ry::2, rx::2]))
                ms.append(emb(m4[:, rz::2, ry::2, rx::2])[0])
    sh = ((0, 0), (0, 0), (0, 0), (1, 0))
    for rz in (0, 1):
        for ry in (0, 1):
            r1 = y4[:, rz::2, ry::2, 1::2]
            m1 = m4[:, rz::2, ry::2, 1::2]
            ys.append(emb(jnp.pad(r1[:, :, :, :-1], sh)))
            ms.append(emb(jnp.pad(m1[:, :, :, :-1], sh))[0])
    return jnp.stack(ys), jnp.stack(ms)


def kernel(voxel_features, coors, batch_size, params):
    D, H, W = _LEV[0]
    feats = voxel_features[:, -4:]
    b = jnp.clip(coors[:, 0], 0, batch_size - 1)
    z, yy, xx = coors[:, 1], coors[:, 2], coors[:, 3]
    dense = jnp.zeros((2, 4, D, H, W), _F32).at[b, :, z, yy, xx].set(feats)
    mask = jnp.zeros((2, 1, D, H, W), _F32).at[b, 0, z, yy, xx].set(1.0)

    p = params
    wa, ba = _prep_w9(p["c0a_w"], p["c0a_g"]), p["c0a_b"][:, None]
    wb, bb = _prep_w9(p["c0b_w"], p["c0b_g"]), p["c0b_b"][:, None]
    dw, subw = {}, {}
    for lvl, dn, sns in ((1, "d0", ["c1a", "c1b"]),
                         (2, "d1", ["c2a", "c2b", "c2c"]),
                         (3, "d2", ["c3a", "c3b", "c3c"])):
        dw[lvl] = (_prep_w27(p[dn + "_w"], p[dn + "_g"]),
                   p[dn + "_b"][:, None])
        ws = []
        for n in sns:
            ws.append(_prep_w9(p[n + "_w"], p[n + "_g"]))
            ws.append(p[n + "_b"][:, None])
        subw[lvl] = ws
    sex = p["ex_g"] / jnp.sqrt(1.0 + 1e-3)
    wex = (p["ex_w"][:, :, 0, 0, 0] * sex[:, None]).astype(_BF)
    bex = p["ex_b"][:, None]
    auxs = [jnp.asarray(_AUX[l]).astype(_BF) for l in range(4)]

    f_l0 = pl.pallas_call(
        _lvl0_body,
        out_shape=jax.ShapeDtypeStruct((16, _Ps[0]), _BF),
        scratch_shapes=[pltpu.VMEM((48, _NT[0]), _BF)])

    def f_down(lvl, c_in, c_out, nc, args):
        return pl.pallas_call(
            _make_down_body(lvl, c_in, c_out, nc),
            out_shape=(jax.ShapeDtypeStruct((c_out, _Ps[lvl]), _BF),
                       jax.ShapeDtypeStruct((8, _Ps[lvl]), _BF)))(*args)

    def f_subm(lvl, c, n_subm, nc, has_ex, args):
        co = 320 if has_ex else c
        odt = _F32 if has_ex else _BF
        return pl.pallas_call(
            _make_subm_body(lvl, c, n_subm, nc, has_ex),
            out_shape=jax.ShapeDtypeStruct((co, _Ps[lvl]), odt),
            scratch_shapes=[pltpu.VMEM((4 * c, _NT[lvl]), _BF)])(*args)

    outs = []
    for bi in range(2):
        xb = jnp.pad(dense[bi], ((0, 0), (1, 1), (1, 1), (0, 0)))
        xb = jnp.pad(xb.reshape(4, -1),
                     ((0, 0), (_Gs[0], _Gs[0]))).astype(_BF)
        mb = jnp.pad(mask[bi], ((0, 0), (1, 1), (1, 1), (0, 0))).reshape(-1)
        aux0 = auxs[0].at[2, _Gs[0]:_Gs[0] + _Ps[0]].set(mb.astype(_BF))
        y0 = f_l0(xb, aux0, wa, ba, wb, bb)

        cur_y, cur_m = y0, mb.astype(_BF)
        for lvl, c_in, c_out, ncd, ncs in ((1, 16, 32, 4, 4),
                                           (2, 32, 64, 2, 2),
                                           (3, 64, 64, 1, 1)):
            yp, mp = _parity_pack(cur_y, cur_m, lvl - 1)
            wd, bd = dw[lvl]
            x1, m8 = f_down(lvl, c_in, c_out, ncd,
                            [yp, mp, auxs[lvl], wd, bd])
            has_ex = lvl == 3
            args = [x1, m8, auxs[lvl]] + subw[lvl]
            if has_ex:
                args += [wex, bex]
            cur_y = f_subm(lvl, c_out, len(subw[lvl]) // 2, ncs, has_ex,
                           args)
            cur_m = m8[0]

        y3 = cur_y.reshape(320, _SD[3], _SH[3], 128)[:, 1:3, 1:17, 0:16]
        outs.append(y3.reshape(640, 16, 16))
    return jnp.stack(outs)
